# Initial kernel scaffold; baseline (speedup 1.0000x reference)
import jax
import jax.numpy as jnp
from jax.experimental import pallas as pl


def kernel(x, rbf, sbf, idx_kj, idx_ji, W_rbf1, W_rbf2, W_sbf1, W_sbf2, W_kj, b_kj, W_ji, b_ji, W_down, W_up, bs0_W1, bs0_b1, bs0_W2, bs0_b2, W_lin, b_lin, as0_W1, as0_b1, as0_W2, as0_b2, as1_W1, as1_b1, as1_W2, as1_b2):
    raise NotImplementedError("write your pallas kernel here")


# speedup vs baseline: 1.0045x; 1.0045x over previous
import functools

import jax
import jax.numpy as jnp
from jax import lax
from jax.experimental import pallas as pl
from jax.experimental.pallas import tpu as pltpu
from jax.experimental.pallas import tpu_sc as plsc

E = 320000
T = 1280000
H = 128
INT = 64

NCORES = 2
NSUB = 16
RANGE = 16384
NRANGES = 20
NPASS = NRANGES // NCORES
TPT = T // NSUB
BS = 4000
NBLK = TPT // BS
K = 256
CAP = 4608
ROWS_PER_TILE = RANGE // NSUB


def _silu(v):
    return v * (1.0 / (1.0 + jnp.exp(-v)))


_BE = 512
_NEB = E // _BE


def _k1_body(x_r, rbf_r, wji_r, bji_r, wkj_r, bkj_r, wr12_r, wdown_r,
             xji_o, xkd_o):
    xb = x_r[...]
    dot = functools.partial(jnp.dot, preferred_element_type=jnp.float32,
                            precision=lax.Precision.HIGHEST)
    xji = _silu(dot(xb, wji_r[...]) + bji_r[...])
    xkj = _silu(dot(xb, wkj_r[...]) + bkj_r[...])
    rh = dot(rbf_r[...], wr12_r[...])
    xkd_o[...] = _silu(dot(xkj * rh, wdown_r[...]))
    xji_o[...] = xji


def _k1(x, rbf, w_ji, b_ji, w_kj, b_kj, w_rbf12, w_down):
    full = lambda s: pl.BlockSpec(s, lambda i: (0, 0))
    return pl.pallas_call(
        _k1_body,
        grid=(_NEB,),
        in_specs=[
            pl.BlockSpec((_BE, H), lambda i: (i, 0)),
            pl.BlockSpec((_BE, 6), lambda i: (i, 0)),
            full((H, H)), full((1, H)), full((H, H)), full((1, H)),
            full((6, H)), full((H, INT)),
        ],
        out_specs=[
            pl.BlockSpec((_BE, H), lambda i: (i, 0)),
            pl.BlockSpec((_BE, INT), lambda i: (i, 0)),
        ],
        out_shape=[
            jax.ShapeDtypeStruct((E, H), jnp.float32),
            jax.ShapeDtypeStruct((E, INT), jnp.float32),
        ],
    )(x, rbf, w_ji, b_ji, w_kj, b_kj, w_rbf12, w_down)


_BT = 2048
_NTB = T // _BT


def _k2_body(sbf_r, w_r, out_r):
    out_r[...] = jnp.dot(sbf_r[...], w_r[...],
                         preferred_element_type=jnp.float32,
                         precision=lax.Precision.HIGHEST)


def _k2(sbf, w_sbf12):
    return pl.pallas_call(
        _k2_body,
        grid=(_NTB,),
        in_specs=[
            pl.BlockSpec((_BT, 42), lambda i: (i, 0)),
            pl.BlockSpec((42, INT), lambda i: (0, 0)),
        ],
        out_specs=pl.BlockSpec((_BT, INT), lambda i: (i, 0)),
        out_shape=jax.ShapeDtypeStruct((T, INT), jnp.float32),
    )(sbf, w_sbf12)


def _k3_body(sbfh, idx_kj, idx_ji, xkd, out,
             ji_blk, kj_blk, t_list, kj_list, ji_list,
             t_idx, kj_idx, ji_idx, sbf_buf, xkj_buf, zbuf, acc, sem1, sem2):
    c = lax.axis_index("c")
    s = lax.axis_index("s")
    iota16 = lax.iota(jnp.int32, 16)
    zero16 = jnp.zeros((16,), jnp.float32)

    def zb(i, _):
        for q in range(INT // 16):
            zbuf[i, pl.ds(q * 16, 16)] = zero16
        return 0
    lax.fori_loop(0, zbuf.shape[0], zb, 0)

    def process_chunk(ci, _):
        def cpidx(i, _):
            sl_d = pl.ds(i * 16, 16)
            sl_s = pl.ds(ci * K + i * 16, 16)
            t_idx[sl_d] = t_list[sl_s]
            kj_idx[sl_d] = kj_list[sl_s]
            ji_idx[sl_d] = ji_list[sl_s]
            return 0
        lax.fori_loop(0, K // 16, cpidx, 0, unroll=4)
        cp1 = pltpu.async_copy(
            sbfh.at[plsc.Indices(t_idx, ignored_value=-1)], sbf_buf, sem1)
        cp2 = pltpu.async_copy(
            xkd.at[plsc.Indices(kj_idx, ignored_value=-1)], xkj_buf, sem2)
        cp1.wait()
        cp2.wait()

        def mul(r, _):
            for q in range(INT // 16):
                sl = pl.ds(q * 16, 16)
                xkj_buf[r, sl] = xkj_buf[r, sl] * sbf_buf[r, sl]
            return 0
        lax.fori_loop(0, K, mul, 0, unroll=4)
        pltpu.sync_copy(
            xkj_buf, acc.at[plsc.Indices(ji_idx, ignored_value=-1)],
            add=True)
        return 0

    tile_t0 = s * TPT

    def pass_body(p, _):
        base = (p * NCORES + c) * RANGE

        def zc(i, _):
            pltpu.sync_copy(
                zbuf, acc.at[pl.ds(s * ROWS_PER_TILE + i * zbuf.shape[0],
                                   zbuf.shape[0])])
            return 0
        lax.fori_loop(0, ROWS_PER_TILE // zbuf.shape[0], zc, 0)
        plsc.subcore_barrier()

        def block_body(b, cnt):
            blk0 = tile_t0 + b * BS
            pltpu.sync_copy(idx_ji.at[pl.ds(blk0, BS)], ji_blk)
            pltpu.sync_copy(idx_kj.at[pl.ds(blk0, BS)], kj_blk)

            def comp(v, cnt):
                ji = ji_blk[pl.ds(v * 16, 16)]
                kj = kj_blk[pl.ds(v * 16, 16)]
                rel = ji - base
                m = (rel >= 0) & (rel < RANGE)
                mi = jnp.where(m, 1, 0).astype(jnp.int32)
                pos = cnt + plsc.cumsum(mi) - 1
                tv = blk0 + v * 16 + iota16
                plsc.store_scatter(t_list, [pos], tv, mask=m)
                plsc.store_scatter(kj_list, [pos], kj, mask=m)
                plsc.store_scatter(ji_list, [pos], rel, mask=m)
                return cnt + jnp.sum(mi)
            cnt = lax.fori_loop(0, BS // 16, comp, cnt)

            nfull = cnt // K
            lax.fori_loop(0, nfull, process_chunk, 0)

            def shift(i, _):
                sl_d = pl.ds(i * 16, 16)
                sl_s = pl.ds(nfull * K + i * 16, 16)
                tv = t_list[sl_s]
                kv = kj_list[sl_s]
                jv = ji_list[sl_s]
                t_list[sl_d] = tv
                kj_list[sl_d] = kv
                ji_list[sl_d] = jv
                return 0
            lax.fori_loop(0, K // 16, shift, 0, unroll=4)
            return cnt - nfull * K
        cnt = lax.fori_loop(0, NBLK, block_body, jnp.int32(0))

        nch = (cnt + (K - 1)) // K
        pend = nch * K
        start = (cnt // 16) * 16
        negs = jnp.full((16,), -1, jnp.int32)

        def padb(i, _):
            posp = start + i * 16 + iota16
            mp_ = (posp >= cnt) & (posp < pend)
            plsc.store_scatter(t_list, [posp], negs, mask=mp_)
            plsc.store_scatter(kj_list, [posp], negs, mask=mp_)
            plsc.store_scatter(ji_list, [posp], negs, mask=mp_)
            return 0
        lax.fori_loop(0, (pend - start + 15) // 16, padb, 0)
        lax.fori_loop(0, nch, process_chunk, 0)

        plsc.subcore_barrier()
        pltpu.sync_copy(
            acc.at[pl.ds(s * ROWS_PER_TILE, ROWS_PER_TILE)],
            out.at[pl.ds(base + s * ROWS_PER_TILE, ROWS_PER_TILE)])
        plsc.subcore_barrier()
        return 0

    lax.fori_loop(0, NPASS, pass_body, 0)


def _k3(sbfh, idx_kj, idx_ji, xkd):
    mesh = plsc.VectorSubcoreMesh(
        core_axis_name="c", subcore_axis_name="s",
        num_cores=NCORES, num_subcores=NSUB)
    kern = pl.kernel(
        _k3_body,
        out_type=jax.ShapeDtypeStruct((NRANGES * RANGE, INT), jnp.float32),
        mesh=mesh,
        compiler_params=pltpu.CompilerParams(
            needs_layout_passes=False, use_tc_tiling_on_sc=False),
        scratch_types=[
            pltpu.VMEM((BS,), jnp.int32),
            pltpu.VMEM((BS,), jnp.int32),
            pltpu.VMEM((CAP,), jnp.int32),
            pltpu.VMEM((CAP,), jnp.int32),
            pltpu.VMEM((CAP,), jnp.int32),
            pltpu.VMEM((K,), jnp.int32),
            pltpu.VMEM((K,), jnp.int32),
            pltpu.VMEM((K,), jnp.int32),
            pltpu.VMEM((K, INT), jnp.float32),
            pltpu.VMEM((K, INT), jnp.float32),
            pltpu.VMEM((32, INT), jnp.float32),
            pltpu.VMEM_SHARED((RANGE, INT), jnp.float32),
            pltpu.SemaphoreType.DMA,
            pltpu.SemaphoreType.DMA,
        ],
    )
    return kern(sbfh, idx_kj, idx_ji, xkd)


def _k4_body(seg_r, xji_r, x_r, wup_r,
             bw1_r, bb1_r, bw2_r, bb2_r, wl_r, bl_r,
             aw1_r, ab1_r, aw2_r, ab2_r, cw1_r, cb1_r, cw2_r, cb2_r,
             out_r):
    dot = functools.partial(jnp.dot, preferred_element_type=jnp.float32,
                            precision=lax.Precision.HIGHEST)
    h = xji_r[...] + _silu(dot(seg_r[...], wup_r[...]))
    h = h + _silu(dot(_silu(dot(h, bw1_r[...]) + bb1_r[...]), bw2_r[...])
                  + bb2_r[...])
    h = _silu(dot(h, wl_r[...]) + bl_r[...]) + x_r[...]
    h = h + _silu(dot(_silu(dot(h, aw1_r[...]) + ab1_r[...]), aw2_r[...])
                  + ab2_r[...])
    h = h + _silu(dot(_silu(dot(h, cw1_r[...]) + cb1_r[...]), cw2_r[...])
                  + cb2_r[...])
    out_r[...] = h


def _k4(seg_ext, xji, x, w_up, bs0_W1, bs0_b1, bs0_W2, bs0_b2,
        w_lin, b_lin, as0_W1, as0_b1, as0_W2, as0_b2,
        as1_W1, as1_b1, as1_W2, as1_b2):
    full = lambda s: pl.BlockSpec(s, lambda i: (0, 0))
    wspec = full((H, H))
    bspec = full((1, H))
    return pl.pallas_call(
        _k4_body,
        grid=(_NEB,),
        in_specs=[
            pl.BlockSpec((_BE, INT), lambda i: (i, 0)),
            pl.BlockSpec((_BE, H), lambda i: (i, 0)),
            pl.BlockSpec((_BE, H), lambda i: (i, 0)),
            full((INT, H)),
            wspec, bspec, wspec, bspec, wspec, bspec,
            wspec, bspec, wspec, bspec, wspec, bspec, wspec, bspec,
        ],
        out_specs=pl.BlockSpec((_BE, H), lambda i: (i, 0)),
        out_shape=jax.ShapeDtypeStruct((E, H), jnp.float32),
    )(seg_ext, xji, x, w_up, bs0_W1, bs0_b1, bs0_W2, bs0_b2,
      w_lin, b_lin, as0_W1, as0_b1, as0_W2, as0_b2,
      as1_W1, as1_b1, as1_W2, as1_b2)


def kernel(x, rbf, sbf, idx_kj, idx_ji, W_rbf1, W_rbf2, W_sbf1, W_sbf2,
           W_kj, b_kj, W_ji, b_ji, W_down, W_up, bs0_W1, bs0_b1, bs0_W2,
           bs0_b2, W_lin, b_lin, as0_W1, as0_b1, as0_W2, as0_b2,
           as1_W1, as1_b1, as1_W2, as1_b2):
    idx_kj = idx_kj.astype(jnp.int32)
    idx_ji = idx_ji.astype(jnp.int32)
    w_rbf12 = W_rbf1 @ W_rbf2
    w_sbf12 = W_sbf1 @ W_sbf2
    r2 = lambda b: b.reshape(1, -1)

    xji, xkd = _k1(x, rbf, W_ji, r2(b_ji), W_kj, r2(b_kj), w_rbf12, W_down)
    sbfh = _k2(sbf, w_sbf12)
    seg_ext = _k3(sbfh, idx_kj, idx_ji, xkd)
    return _k4(seg_ext, xji, x, W_up, bs0_W1, r2(bs0_b1), bs0_W2, r2(bs0_b2),
               W_lin, r2(b_lin), as0_W1, r2(as0_b1), as0_W2, r2(as0_b2),
               as1_W1, r2(as1_b1), as1_W2, r2(as1_b2))


# speedup vs baseline: 1.4993x; 1.4926x over previous
import functools

import jax
import jax.numpy as jnp
from jax import lax
from jax.experimental import pallas as pl
from jax.experimental.pallas import tpu as pltpu
from jax.experimental.pallas import tpu_sc as plsc

E = 320000
T = 1280000
H = 128
INT = 64

NCORES = 2
NSUB = 16
RANGE = 16384
NRANGES = 20
NPASS = NRANGES // NCORES
TPT = T // NSUB
BS = 4000
NBLK = TPT // BS
K = 256
CAP = 4608
ROWS_PER_TILE = RANGE // NSUB


def _silu(v):
    return v * (1.0 / (1.0 + jnp.exp(-v)))


_BE = 512
_NEB = E // _BE


def _k1_body(x_r, rbf_r, wji_r, bji_r, wkj_r, bkj_r, wr12_r, wdown_r,
             xji_o, xkd_o):
    xb = x_r[...]
    dot = functools.partial(jnp.dot, preferred_element_type=jnp.float32)
    xji = _silu(dot(xb, wji_r[...]) + bji_r[...])
    xkj = _silu(dot(xb, wkj_r[...]) + bkj_r[...])
    rh = dot(rbf_r[...], wr12_r[...])
    xkd_o[...] = _silu(dot(xkj * rh, wdown_r[...]))
    xji_o[...] = xji


def _k1(x, rbf, w_ji, b_ji, w_kj, b_kj, w_rbf12, w_down):
    full = lambda s: pl.BlockSpec(s, lambda i: (0, 0))
    return pl.pallas_call(
        _k1_body,
        grid=(_NEB,),
        in_specs=[
            pl.BlockSpec((_BE, H), lambda i: (i, 0)),
            pl.BlockSpec((_BE, 6), lambda i: (i, 0)),
            full((H, H)), full((1, H)), full((H, H)), full((1, H)),
            full((6, H)), full((H, INT)),
        ],
        out_specs=[
            pl.BlockSpec((_BE, H), lambda i: (i, 0)),
            pl.BlockSpec((_BE, INT), lambda i: (i, 0)),
        ],
        out_shape=[
            jax.ShapeDtypeStruct((E, H), jnp.float32),
            jax.ShapeDtypeStruct((E, INT), jnp.float32),
        ],
    )(x, rbf, w_ji, b_ji, w_kj, b_kj, w_rbf12, w_down)


_BT = 5120
_NTB = T // _BT


def _k2_body(sbf_r, w_r, out_r):
    out_r[...] = jnp.dot(sbf_r[...], w_r[...],
                         preferred_element_type=jnp.float32,
)


def _k2(sbf, w_sbf12):
    return pl.pallas_call(
        _k2_body,
        grid=(_NTB,),
        in_specs=[
            pl.BlockSpec((_BT, 42), lambda i: (i, 0)),
            pl.BlockSpec((42, INT), lambda i: (0, 0)),
        ],
        out_specs=pl.BlockSpec((_BT, INT), lambda i: (i, 0)),
        out_shape=jax.ShapeDtypeStruct((T, INT), jnp.float32),
    )(sbf, w_sbf12)


def _k3_body(sbfh, idx_kj, idx_ji, xkd, out,
             ji_blk, kj_blk, t_list, kj_list, ji_list,
             t_idx, kj_idx, ji_idx, sbf_buf, xkj_buf, zbuf, acc, sem1, sem2):
    c = lax.axis_index("c")
    s = lax.axis_index("s")
    iota16 = lax.iota(jnp.int32, 16)
    zero16 = jnp.zeros((16,), jnp.float32)

    def zb(i, _):
        for q in range(INT // 16):
            zbuf[i, pl.ds(q * 16, 16)] = zero16
        return 0
    lax.fori_loop(0, zbuf.shape[0], zb, 0)

    def process_chunk(ci, _):
        def cpidx(i, _):
            sl_d = pl.ds(i * 16, 16)
            sl_s = pl.ds(ci * K + i * 16, 16)
            t_idx[sl_d] = t_list[sl_s]
            kj_idx[sl_d] = kj_list[sl_s]
            ji_idx[sl_d] = ji_list[sl_s]
            return 0
        lax.fori_loop(0, K // 16, cpidx, 0, unroll=4)
        cp1 = pltpu.async_copy(
            sbfh.at[plsc.Indices(t_idx, ignored_value=-1)], sbf_buf, sem1)
        cp2 = pltpu.async_copy(
            xkd.at[plsc.Indices(kj_idx, ignored_value=-1)], xkj_buf, sem2)
        cp1.wait()
        cp2.wait()

        def mul(r, _):
            for q in range(INT // 16):
                sl = pl.ds(q * 16, 16)
                xkj_buf[r, sl] = xkj_buf[r, sl] * sbf_buf[r, sl]
            return 0
        lax.fori_loop(0, K, mul, 0, unroll=4)
        pltpu.sync_copy(
            xkj_buf, acc.at[plsc.Indices(ji_idx, ignored_value=-1)],
            add=True)
        return 0

    tile_t0 = s * TPT

    def pass_body(p, _):
        base = (p * NCORES + c) * RANGE

        def zc(i, _):
            pltpu.sync_copy(
                zbuf, acc.at[pl.ds(s * ROWS_PER_TILE + i * zbuf.shape[0],
                                   zbuf.shape[0])])
            return 0
        lax.fori_loop(0, ROWS_PER_TILE // zbuf.shape[0], zc, 0)
        plsc.subcore_barrier()

        def block_body(b, cnt):
            blk0 = tile_t0 + b * BS
            pltpu.sync_copy(idx_ji.at[pl.ds(blk0, BS)], ji_blk)
            pltpu.sync_copy(idx_kj.at[pl.ds(blk0, BS)], kj_blk)

            def comp(v, cnt):
                ji = ji_blk[pl.ds(v * 16, 16)]
                kj = kj_blk[pl.ds(v * 16, 16)]
                rel = ji - base
                m = (rel >= 0) & (rel < RANGE)
                mi = jnp.where(m, 1, 0).astype(jnp.int32)
                pos = cnt + plsc.cumsum(mi) - 1
                tv = blk0 + v * 16 + iota16
                plsc.store_scatter(t_list, [pos], tv, mask=m)
                plsc.store_scatter(kj_list, [pos], kj, mask=m)
                plsc.store_scatter(ji_list, [pos], rel, mask=m)
                return cnt + jnp.sum(mi)
            cnt = lax.fori_loop(0, BS // 16, comp, cnt)

            nfull = cnt // K
            lax.fori_loop(0, nfull, process_chunk, 0)

            def shift(i, _):
                sl_d = pl.ds(i * 16, 16)
                sl_s = pl.ds(nfull * K + i * 16, 16)
                tv = t_list[sl_s]
                kv = kj_list[sl_s]
                jv = ji_list[sl_s]
                t_list[sl_d] = tv
                kj_list[sl_d] = kv
                ji_list[sl_d] = jv
                return 0
            lax.fori_loop(0, K // 16, shift, 0, unroll=4)
            return cnt - nfull * K
        cnt = lax.fori_loop(0, NBLK, block_body, jnp.int32(0))

        nch = (cnt + (K - 1)) // K
        pend = nch * K
        start = (cnt // 16) * 16
        negs = jnp.full((16,), -1, jnp.int32)

        def padb(i, _):
            posp = start + i * 16 + iota16
            mp_ = (posp >= cnt) & (posp < pend)
            plsc.store_scatter(t_list, [posp], negs, mask=mp_)
            plsc.store_scatter(kj_list, [posp], negs, mask=mp_)
            plsc.store_scatter(ji_list, [posp], negs, mask=mp_)
            return 0
        lax.fori_loop(0, (pend - start + 15) // 16, padb, 0)
        lax.fori_loop(0, nch, process_chunk, 0)

        plsc.subcore_barrier()
        pltpu.sync_copy(
            acc.at[pl.ds(s * ROWS_PER_TILE, ROWS_PER_TILE)],
            out.at[pl.ds(base + s * ROWS_PER_TILE, ROWS_PER_TILE)])
        plsc.subcore_barrier()
        return 0

    lax.fori_loop(0, NPASS, pass_body, 0)


def _k3(sbfh, idx_kj, idx_ji, xkd):
    mesh = plsc.VectorSubcoreMesh(
        core_axis_name="c", subcore_axis_name="s",
        num_cores=NCORES, num_subcores=NSUB)
    kern = pl.kernel(
        _k3_body,
        out_type=jax.ShapeDtypeStruct((NRANGES * RANGE, INT), jnp.float32),
        mesh=mesh,
        compiler_params=pltpu.CompilerParams(
            needs_layout_passes=False, use_tc_tiling_on_sc=False),
        scratch_types=[
            pltpu.VMEM((BS,), jnp.int32),
            pltpu.VMEM((BS,), jnp.int32),
            pltpu.VMEM((CAP,), jnp.int32),
            pltpu.VMEM((CAP,), jnp.int32),
            pltpu.VMEM((CAP,), jnp.int32),
            pltpu.VMEM((K,), jnp.int32),
            pltpu.VMEM((K,), jnp.int32),
            pltpu.VMEM((K,), jnp.int32),
            pltpu.VMEM((K, INT), jnp.float32),
            pltpu.VMEM((K, INT), jnp.float32),
            pltpu.VMEM((32, INT), jnp.float32),
            pltpu.VMEM_SHARED((RANGE, INT), jnp.float32),
            pltpu.SemaphoreType.DMA,
            pltpu.SemaphoreType.DMA,
        ],
    )
    return kern(sbfh, idx_kj, idx_ji, xkd)


def _k4_body(seg_r, xji_r, x_r, wup_r,
             bw1_r, bb1_r, bw2_r, bb2_r, wl_r, bl_r,
             aw1_r, ab1_r, aw2_r, ab2_r, cw1_r, cb1_r, cw2_r, cb2_r,
             out_r):
    dot = functools.partial(jnp.dot, preferred_element_type=jnp.float32)
    h = xji_r[...] + _silu(dot(seg_r[...], wup_r[...]))
    h = h + _silu(dot(_silu(dot(h, bw1_r[...]) + bb1_r[...]), bw2_r[...])
                  + bb2_r[...])
    h = _silu(dot(h, wl_r[...]) + bl_r[...]) + x_r[...]
    h = h + _silu(dot(_silu(dot(h, aw1_r[...]) + ab1_r[...]), aw2_r[...])
                  + ab2_r[...])
    h = h + _silu(dot(_silu(dot(h, cw1_r[...]) + cb1_r[...]), cw2_r[...])
                  + cb2_r[...])
    out_r[...] = h


def _k4(seg_ext, xji, x, w_up, bs0_W1, bs0_b1, bs0_W2, bs0_b2,
        w_lin, b_lin, as0_W1, as0_b1, as0_W2, as0_b2,
        as1_W1, as1_b1, as1_W2, as1_b2):
    full = lambda s: pl.BlockSpec(s, lambda i: (0, 0))
    wspec = full((H, H))
    bspec = full((1, H))
    return pl.pallas_call(
        _k4_body,
        grid=(_NEB,),
        in_specs=[
            pl.BlockSpec((_BE, INT), lambda i: (i, 0)),
            pl.BlockSpec((_BE, H), lambda i: (i, 0)),
            pl.BlockSpec((_BE, H), lambda i: (i, 0)),
            full((INT, H)),
            wspec, bspec, wspec, bspec, wspec, bspec,
            wspec, bspec, wspec, bspec, wspec, bspec, wspec, bspec,
        ],
        out_specs=pl.BlockSpec((_BE, H), lambda i: (i, 0)),
        out_shape=jax.ShapeDtypeStruct((E, H), jnp.float32),
    )(seg_ext, xji, x, w_up, bs0_W1, bs0_b1, bs0_W2, bs0_b2,
      w_lin, b_lin, as0_W1, as0_b1, as0_W2, as0_b2,
      as1_W1, as1_b1, as1_W2, as1_b2)


def kernel(x, rbf, sbf, idx_kj, idx_ji, W_rbf1, W_rbf2, W_sbf1, W_sbf2,
           W_kj, b_kj, W_ji, b_ji, W_down, W_up, bs0_W1, bs0_b1, bs0_W2,
           bs0_b2, W_lin, b_lin, as0_W1, as0_b1, as0_W2, as0_b2,
           as1_W1, as1_b1, as1_W2, as1_b2):
    idx_kj = idx_kj.astype(jnp.int32)
    idx_ji = idx_ji.astype(jnp.int32)
    w_rbf12 = W_rbf1 @ W_rbf2
    w_sbf12 = W_sbf1 @ W_sbf2
    r2 = lambda b: b.reshape(1, -1)

    xji, xkd = _k1(x, rbf, W_ji, r2(b_ji), W_kj, r2(b_kj), w_rbf12, W_down)
    sbfh = _k2(sbf, w_sbf12)
    seg_ext = _k3(sbfh, idx_kj, idx_ji, xkd)
    return _k4(seg_ext, xji, x, W_up, bs0_W1, r2(bs0_b1), bs0_W2, r2(bs0_b2),
               W_lin, r2(b_lin), as0_W1, r2(as0_b1), as0_W2, r2(as0_b2),
               as1_W1, r2(as1_b1), as1_W2, r2(as1_b2))
